# baseline (speedup 1.0000x reference)
import jax
import jax.numpy as jnp
from jax.experimental import pallas as pl
from jax.experimental.pallas import tpu as pltpu

_B = 1024
_L = 64
_KBIG = 10000
_KC = 512
_RING = 8
_CHUNKS = tuple((k * _KC, min(_KC, _KBIG - k * _KC))
                for k in range((_KBIG + _KC - 1) // _KC))
_NCH = len(_CHUNKS)


def _body(xd_hbm, xm_hbm, xc_hbm, xp_hbm, xco_hbm,
          wd_ref, wm_ref, wc_ref, wp_ref, wco_ref,
          out_ref,
          bd, bm, bc, bp, bco, sems, ssem):
    dn = (((1,), (0,)), ((), ()))

    def start_big(hbm, buf, col, k):
        off, width = _CHUNKS[k]
        pltpu.make_async_copy(
            hbm.at[pl.ds(off, width), :],
            buf.at[k % _RING, pl.ds(0, width), :],
            sems.at[k % _RING, col],
        ).start()

    def wait_big(hbm, buf, col, k):
        off, width = _CHUNKS[k]
        pltpu.make_async_copy(
            hbm.at[pl.ds(off, width), :],
            buf.at[k % _RING, pl.ds(0, width), :],
            sems.at[k % _RING, col],
        ).wait()

    pltpu.make_async_copy(xc_hbm, bc, ssem.at[0]).start()
    pltpu.make_async_copy(xco_hbm, bco, ssem.at[1]).start()
    pltpu.make_async_copy(xd_hbm, bd, ssem.at[2]).start()
    for k in range(_RING):
        start_big(xm_hbm, bm, 0, k)
        start_big(xp_hbm, bp, 1, k)

    def part(w, xi):
        y = jax.lax.dot_general(
            w.astype(jnp.bfloat16), xi.astype(jnp.bfloat16), dn,
            preferred_element_type=jnp.float32)
        s = jnp.sum(xi, axis=0, keepdims=True)
        return y, s

    ym = sm = yp = sp = None
    for k in range(_NCH):
        off, width = _CHUNKS[k]
        wait_big(xm_hbm, bm, 0, k)
        pm, qm = part(wm_ref[:, pl.ds(off, width)],
                      bm[k % _RING, pl.ds(0, width), :])
        ym = pm if ym is None else ym + pm
        sm = qm if sm is None else sm + qm
        if k + _RING < _NCH:
            start_big(xm_hbm, bm, 0, k + _RING)

        wait_big(xp_hbm, bp, 1, k)
        pp, qp = part(wp_ref[:, pl.ds(off, width)],
                      bp[k % _RING, pl.ds(0, width), :])
        yp = pp if yp is None else yp + pp
        sp = qp if sp is None else sp + qp
        if k + _RING < _NCH:
            start_big(xp_hbm, bp, 1, k + _RING)

        if k == 2:
            pltpu.make_async_copy(xc_hbm, bc, ssem.at[0]).wait()
            yc, sc = part(wc_ref[...], bc[...])
            pltpu.make_async_copy(xco_hbm, bco, ssem.at[1]).wait()
            yco, sco = part(wco_ref[...], bco[...])
            pltpu.make_async_copy(xd_hbm, bd, ssem.at[2]).wait()
            yd, sd = part(wd_ref[...], bd[...])

    def mean_div(y, s):
        nz = s != 0
        sf = jnp.where(nz, s, 1).astype(jnp.float32)
        return jnp.where(nz, y / sf, y)

    yd = mean_div(yd, sd)
    yd = mean_div(yd, sm)
    yc = mean_div(yc, sc)
    yp = mean_div(yp, sp)
    yco = mean_div(yco, sco)

    out_ref[...] = jnp.concatenate((yd, ym, yc, yp, yco), axis=0)


def kernel(decade_idxs, movie_idxs, category_idxs, person_idxs, company_idxs,
           W_decade, W_movie, W_category, W_person, W_company):
    xd, xm, xc, xp, xco = (x.T for x in (
        decade_idxs, movie_idxs, category_idxs, person_idxs, company_idxs))

    any_spec = pl.BlockSpec(memory_space=pl.ANY)
    in_specs = [
        any_spec, any_spec, any_spec, any_spec, any_spec,
        pl.BlockSpec((_L, 16), lambda: (0, 0)),
        pl.BlockSpec((_L, _KBIG), lambda: (0, 0)),
        pl.BlockSpec((_L, 1000), lambda: (0, 0)),
        pl.BlockSpec((_L, _KBIG), lambda: (0, 0)),
        pl.BlockSpec((_L, 1000), lambda: (0, 0)),
    ]
    scratch_shapes = [
        pltpu.VMEM((16, _B), jnp.int32),
        pltpu.VMEM((_RING, _KC, _B), jnp.int32),
        pltpu.VMEM((1000, _B), jnp.int32),
        pltpu.VMEM((_RING, _KC, _B), jnp.int32),
        pltpu.VMEM((1000, _B), jnp.int32),
        pltpu.SemaphoreType.DMA((_RING, 2)),
        pltpu.SemaphoreType.DMA((3,)),
    ]
    out_t = pl.pallas_call(
        _body,
        in_specs=in_specs,
        out_specs=pl.BlockSpec((5 * _L, _B), lambda: (0, 0)),
        out_shape=jax.ShapeDtypeStruct((5 * _L, _B), jnp.float32),
        scratch_shapes=scratch_shapes,
    )(xd, xm, xc, xp, xco,
      W_decade, W_movie, W_category, W_person, W_company)
    return out_t.T


# speedup vs baseline: 1.0136x; 1.0136x over previous
import jax
import jax.numpy as jnp
from jax.experimental import pallas as pl
from jax.experimental.pallas import tpu as pltpu

_B = 1024
_L = 64
_KBIG = 10000
_KC = 1024
_RING = 5
_CHUNKS = tuple((k * _KC, min(_KC, _KBIG - k * _KC))
                for k in range((_KBIG + _KC - 1) // _KC))
_NCH = len(_CHUNKS)


def _body(xd_hbm, xm_hbm, xc_hbm, xp_hbm, xco_hbm,
          wd_ref, wm_ref, wc_ref, wp_ref, wco_ref,
          out_ref,
          bd, bm, bc, bp, bco, sems, ssem):
    dn = (((1,), (0,)), ((), ()))

    def start_big(hbm, buf, col, k):
        off, width = _CHUNKS[k]
        pltpu.make_async_copy(
            hbm.at[pl.ds(off, width), :],
            buf.at[k % _RING, pl.ds(0, width), :],
            sems.at[k % _RING, col],
        ).start()

    def wait_big(hbm, buf, col, k):
        off, width = _CHUNKS[k]
        pltpu.make_async_copy(
            hbm.at[pl.ds(off, width), :],
            buf.at[k % _RING, pl.ds(0, width), :],
            sems.at[k % _RING, col],
        ).wait()

    pltpu.make_async_copy(xc_hbm, bc, ssem.at[0]).start()
    pltpu.make_async_copy(xco_hbm, bco, ssem.at[1]).start()
    pltpu.make_async_copy(xd_hbm, bd, ssem.at[2]).start()
    for k in range(_RING):
        start_big(xm_hbm, bm, 0, k)
        start_big(xp_hbm, bp, 1, k)

    def part(w, xi):
        y = jax.lax.dot_general(
            w.astype(jnp.bfloat16), xi.astype(jnp.bfloat16), dn,
            preferred_element_type=jnp.float32)
        s = jnp.sum(xi, axis=0, keepdims=True)
        return y, s

    ym = sm = yp = sp = None
    for k in range(_NCH):
        off, width = _CHUNKS[k]
        wait_big(xm_hbm, bm, 0, k)
        pm, qm = part(wm_ref[:, pl.ds(off, width)],
                      bm[k % _RING, pl.ds(0, width), :])
        ym = pm if ym is None else ym + pm
        sm = qm if sm is None else sm + qm
        if k + _RING < _NCH:
            start_big(xm_hbm, bm, 0, k + _RING)

        wait_big(xp_hbm, bp, 1, k)
        pp, qp = part(wp_ref[:, pl.ds(off, width)],
                      bp[k % _RING, pl.ds(0, width), :])
        yp = pp if yp is None else yp + pp
        sp = qp if sp is None else sp + qp
        if k + _RING < _NCH:
            start_big(xp_hbm, bp, 1, k + _RING)

        if k == 2:
            pltpu.make_async_copy(xc_hbm, bc, ssem.at[0]).wait()
            yc, sc = part(wc_ref[...], bc[...])
            pltpu.make_async_copy(xco_hbm, bco, ssem.at[1]).wait()
            yco, sco = part(wco_ref[...], bco[...])
            pltpu.make_async_copy(xd_hbm, bd, ssem.at[2]).wait()
            yd, sd = part(wd_ref[...], bd[...])

    def mean_div(y, s):
        nz = s != 0
        sf = jnp.where(nz, s, 1).astype(jnp.float32)
        return jnp.where(nz, y / sf, y)

    yd = mean_div(yd, sd)
    yd = mean_div(yd, sm)
    yc = mean_div(yc, sc)
    yp = mean_div(yp, sp)
    yco = mean_div(yco, sco)

    out_ref[...] = jnp.concatenate((yd, ym, yc, yp, yco), axis=0)


def kernel(decade_idxs, movie_idxs, category_idxs, person_idxs, company_idxs,
           W_decade, W_movie, W_category, W_person, W_company):
    xd, xm, xc, xp, xco = (x.T for x in (
        decade_idxs, movie_idxs, category_idxs, person_idxs, company_idxs))

    any_spec = pl.BlockSpec(memory_space=pl.ANY)
    in_specs = [
        any_spec, any_spec, any_spec, any_spec, any_spec,
        pl.BlockSpec((_L, 16), lambda: (0, 0)),
        pl.BlockSpec((_L, _KBIG), lambda: (0, 0)),
        pl.BlockSpec((_L, 1000), lambda: (0, 0)),
        pl.BlockSpec((_L, _KBIG), lambda: (0, 0)),
        pl.BlockSpec((_L, 1000), lambda: (0, 0)),
    ]
    scratch_shapes = [
        pltpu.VMEM((16, _B), jnp.int32),
        pltpu.VMEM((_RING, _KC, _B), jnp.int32),
        pltpu.VMEM((1000, _B), jnp.int32),
        pltpu.VMEM((_RING, _KC, _B), jnp.int32),
        pltpu.VMEM((1000, _B), jnp.int32),
        pltpu.SemaphoreType.DMA((_RING, 2)),
        pltpu.SemaphoreType.DMA((3,)),
    ]
    out_t = pl.pallas_call(
        _body,
        in_specs=in_specs,
        out_specs=pl.BlockSpec((5 * _L, _B), lambda: (0, 0)),
        out_shape=jax.ShapeDtypeStruct((5 * _L, _B), jnp.float32),
        scratch_shapes=scratch_shapes,
        compiler_params=pltpu.CompilerParams(
            vmem_limit_bytes=64 * 1024 * 1024),
    )(xd, xm, xc, xp, xco,
      W_decade, W_movie, W_category, W_person, W_company)
    return out_t.T


# speedup vs baseline: 1.0197x; 1.0061x over previous
import jax
import jax.numpy as jnp
from jax.experimental import pallas as pl
from jax.experimental.pallas import tpu as pltpu

_B = 1024
_L = 64
_KBIG = 10000
_KC = 1024
_RING = 4
_CHUNKS = tuple((k * _KC, min(_KC, _KBIG - k * _KC))
                for k in range((_KBIG + _KC - 1) // _KC))
_NCH = len(_CHUNKS)


def _body(xd_hbm, xm_hbm, xc_hbm, xp_hbm, xco_hbm,
          wd_ref, wm_ref, wc_ref, wp_ref, wco_ref,
          out_ref,
          bd, bm, bc, bp, bco, sems, ssem):
    dn = (((1,), (0,)), ((), ()))

    def start_big(hbm, buf, col, k):
        off, width = _CHUNKS[k]
        pltpu.make_async_copy(
            hbm.at[pl.ds(off, width), :],
            buf.at[k % _RING, pl.ds(0, width), :],
            sems.at[k % _RING, col],
        ).start()

    def wait_big(hbm, buf, col, k):
        off, width = _CHUNKS[k]
        pltpu.make_async_copy(
            hbm.at[pl.ds(off, width), :],
            buf.at[k % _RING, pl.ds(0, width), :],
            sems.at[k % _RING, col],
        ).wait()

    pltpu.make_async_copy(xc_hbm, bc, ssem.at[0]).start()
    pltpu.make_async_copy(xco_hbm, bco, ssem.at[1]).start()
    pltpu.make_async_copy(xd_hbm, bd, ssem.at[2]).start()
    for k in range(_RING):
        start_big(xm_hbm, bm, 0, k)
        start_big(xp_hbm, bp, 1, k)

    def part(w, xi):
        y = jax.lax.dot_general(
            w.astype(jnp.bfloat16), xi.astype(jnp.bfloat16), dn,
            preferred_element_type=jnp.float32)
        s = jnp.sum(xi, axis=0, keepdims=True)
        return y, s

    ym = sm = yp = sp = None
    for k in range(_NCH):
        off, width = _CHUNKS[k]
        wait_big(xm_hbm, bm, 0, k)
        pm, qm = part(wm_ref[:, pl.ds(off, width)],
                      bm[k % _RING, pl.ds(0, width), :])
        ym = pm if ym is None else ym + pm
        sm = qm if sm is None else sm + qm
        if k + _RING < _NCH:
            start_big(xm_hbm, bm, 0, k + _RING)

        wait_big(xp_hbm, bp, 1, k)
        pp, qp = part(wp_ref[:, pl.ds(off, width)],
                      bp[k % _RING, pl.ds(0, width), :])
        yp = pp if yp is None else yp + pp
        sp = qp if sp is None else sp + qp
        if k + _RING < _NCH:
            start_big(xp_hbm, bp, 1, k + _RING)

        if k == 2:
            pltpu.make_async_copy(xc_hbm, bc, ssem.at[0]).wait()
            yc, sc = part(wc_ref[...], bc[...])
            pltpu.make_async_copy(xco_hbm, bco, ssem.at[1]).wait()
            yco, sco = part(wco_ref[...], bco[...])
            pltpu.make_async_copy(xd_hbm, bd, ssem.at[2]).wait()
            yd, sd = part(wd_ref[...], bd[...])

    def mean_div(y, s):
        nz = s != 0
        sf = jnp.where(nz, s, 1).astype(jnp.float32)
        return jnp.where(nz, y / sf, y)

    yd = mean_div(yd, sd)
    yd = mean_div(yd, sm)
    yc = mean_div(yc, sc)
    yp = mean_div(yp, sp)
    yco = mean_div(yco, sco)

    out_ref[...] = jnp.concatenate((yd, ym, yc, yp, yco), axis=0)


def kernel(decade_idxs, movie_idxs, category_idxs, person_idxs, company_idxs,
           W_decade, W_movie, W_category, W_person, W_company):
    xd, xm, xc, xp, xco = (x.T for x in (
        decade_idxs, movie_idxs, category_idxs, person_idxs, company_idxs))

    any_spec = pl.BlockSpec(memory_space=pl.ANY)
    in_specs = [
        any_spec, any_spec, any_spec, any_spec, any_spec,
        pl.BlockSpec((_L, 16), lambda: (0, 0)),
        pl.BlockSpec((_L, _KBIG), lambda: (0, 0)),
        pl.BlockSpec((_L, 1000), lambda: (0, 0)),
        pl.BlockSpec((_L, _KBIG), lambda: (0, 0)),
        pl.BlockSpec((_L, 1000), lambda: (0, 0)),
    ]
    scratch_shapes = [
        pltpu.VMEM((16, _B), jnp.int32),
        pltpu.VMEM((_RING, _KC, _B), jnp.int32),
        pltpu.VMEM((1000, _B), jnp.int32),
        pltpu.VMEM((_RING, _KC, _B), jnp.int32),
        pltpu.VMEM((1000, _B), jnp.int32),
        pltpu.SemaphoreType.DMA((_RING, 2)),
        pltpu.SemaphoreType.DMA((3,)),
    ]
    out_t = pl.pallas_call(
        _body,
        in_specs=in_specs,
        out_specs=pl.BlockSpec((5 * _L, _B), lambda: (0, 0)),
        out_shape=jax.ShapeDtypeStruct((5 * _L, _B), jnp.float32),
        scratch_shapes=scratch_shapes,
    )(xd, xm, xc, xp, xco,
      W_decade, W_movie, W_category, W_person, W_company)
    return out_t.T
